# baseline (speedup 1.0000x reference)
import jax
import jax.numpy as jnp
from jax import lax
from jax.experimental import pallas as pl

_BN = 5632


def _matmul_block(f_ref, x_ref, o_ref):
    f = f_ref[...].astype(jnp.bfloat16)
    x = x_ref[...].astype(jnp.bfloat16)
    o_ref[...] = lax.dot_general(
        f, x, (((1,), (1,)), ((), ())),
        preferred_element_type=jnp.float32)


def kernel(inputs, indexes, features):
    del indexes
    B, D = inputs.shape
    M = features.shape[0]
    out_t = pl.pallas_call(
        _matmul_block,
        grid=(pl.cdiv(M, _BN),),
        in_specs=[
            pl.BlockSpec((_BN, D), lambda i: (i, 0)),
            pl.BlockSpec((B, D), lambda i: (0, 0)),
        ],
        out_specs=pl.BlockSpec((_BN, B), lambda i: (i, 0)),
        out_shape=jax.ShapeDtypeStruct((M, B), jnp.float32),
    )(features, inputs)
    return out_t.T


# speedup vs baseline: 1.0050x; 1.0050x over previous
import jax
import jax.numpy as jnp
from jax import lax
from jax.experimental import pallas as pl

_BN = 5120


def _matmul_block(f_ref, x_ref, o_ref):
    f = f_ref[...].astype(jnp.bfloat16)
    x = x_ref[...].astype(jnp.bfloat16)
    o_ref[...] = lax.dot_general(
        f, x, (((1,), (1,)), ((), ())),
        preferred_element_type=jnp.float32)


def kernel(inputs, indexes, features):
    del indexes
    B, D = inputs.shape
    M = features.shape[0]
    out_t = pl.pallas_call(
        _matmul_block,
        grid=(pl.cdiv(M, _BN),),
        in_specs=[
            pl.BlockSpec((_BN, D), lambda i: (i, 0)),
            pl.BlockSpec((B, D), lambda i: (0, 0)),
        ],
        out_specs=pl.BlockSpec((_BN, B), lambda i: (i, 0)),
        out_shape=jax.ShapeDtypeStruct((M, B), jnp.float32),
    )(features, inputs)
    return out_t.T
